# baseline (speedup 1.0000x reference)
import functools

import jax
import jax.numpy as jnp
from jax import lax
from jax.experimental import pallas as pl
from jax.experimental.pallas import tpu as pltpu
from jax.experimental.pallas import tpu_sc as plsc

N_EXPERTS = 32
N_EMBD = 256
HIDDEN = 4 * N_EMBD
TOPK = 7
NPK = TOPK + 1
T = 2048

BTG = 256
BT = 128
BTC = 256
NBLK = 160
PADTOT = NBLK * BT

_NC, _NS = 2, 16
_NW = _NC * _NS

_INV_SQRT2 = 0.7071067811865476


def _gate_kernel(x_ref, wgt_ref, bgp_ref, rw_ref, idx_ref, w8_ref):
    xb = x_ref[...]
    logits = jnp.dot(xb, wgt_ref[...], preferred_element_type=jnp.float32)
    logits = logits + bgp_ref[...]
    m = jnp.max(logits, axis=1, keepdims=True)
    ex = jnp.exp(logits - m)
    z = jnp.sum(ex, axis=1, keepdims=True)
    p = ex / z
    lane = jax.lax.broadcasted_iota(jnp.int32, (BTG, 128), 1)
    valid = lane < (N_EXPERTS - 1)
    p = jnp.where(valid, p, -1.0)
    pcur = p
    selmask = jnp.zeros((BTG, 128), dtype=jnp.bool_)
    vals, idxs = [], []
    for _ in range(TOPK):
        mj = jnp.max(pcur, axis=1, keepdims=True)
        ismax = pcur == mj
        selidx = jnp.min(jnp.where(ismax, lane, 127), axis=1, keepdims=True)
        mask_j = lane == selidx
        selmask = jnp.logical_or(selmask, mask_j)
        vals.append(mj)
        idxs.append(selidx)
        pcur = jnp.where(mask_j, -1.0, pcur)
    psel = jnp.where(selmask, p, 0.0)
    s = jnp.sum(psel, axis=1, keepdims=True)
    scale = (TOPK / (TOPK + 1.0)) / s
    col0 = jnp.full((BTG, 1), 1.0 / (TOPK + 1.0), dtype=jnp.float32)
    rw_ref[...] = jnp.concatenate([col0, (psel * scale)[:, : N_EXPERTS - 1]],
                                  axis=1)
    idx_ref[...] = jnp.concatenate(
        [jnp.zeros((BTG, 1), jnp.int32)] + [i + 1 for i in idxs], axis=1)
    w8_ref[...] = jnp.concatenate([col0] + [v * scale for v in vals], axis=1)


def _sc_gather(table, idx3, chunk, nchunks, out_rows):
    d = table.shape[1]
    bpw = nchunks * chunk
    mesh = plsc.VectorSubcoreMesh(core_axis_name="c", subcore_axis_name="s")

    @functools.partial(
        pl.kernel,
        mesh=mesh,
        out_type=jax.ShapeDtypeStruct((out_rows, d), jnp.float32),
        scratch_types=[
            pltpu.VMEM((nchunks, chunk), jnp.int32),
            pltpu.VMEM((chunk, d), jnp.float32),
            pltpu.SemaphoreType.DMA,
        ],
    )
    def gather_k(table_hbm, idx_hbm, out_hbm, idx_v, rows_v, sem):
        wid = lax.axis_index("s") * _NC + lax.axis_index("c")
        base = wid * bpw
        pltpu.sync_copy(idx_hbm.at[wid], idx_v)
        for ci in range(nchunks):
            pltpu.async_copy(table_hbm.at[idx_v.at[ci]], rows_v, sem).wait()
            pltpu.sync_copy(rows_v, out_hbm.at[pl.ds(base + ci * chunk, chunk)])

    return gather_k(table, idx3)


def _mlp_kernel(be_ref, nb_ref, x_ref, wfc_ref, bfc_ref, wproj_ref, bproj_ref,
                y_ref):
    b = pl.program_id(0)

    @pl.when(b < nb_ref[0])
    def _():
        xb = x_ref[...].astype(jnp.bfloat16)
        h = jax.lax.dot_general(xb, wfc_ref[0], (((1,), (1,)), ((), ())),
                                preferred_element_type=jnp.float32)
        h = h + bfc_ref[0]
        h = 0.5 * h * (1.0 + jax.lax.erf(h * _INV_SQRT2))
        y = jax.lax.dot_general(h.astype(jnp.bfloat16), wproj_ref[0],
                                (((1,), (1,)), ((), ())),
                                preferred_element_type=jnp.float32)
        y_ref[...] = y + bproj_ref[0]


def _expmap(rb, y, cv):
    xn2 = jnp.sum(rb * rb, axis=-1, keepdims=True)
    sf = 2.0 / (1.0 + cv * xn2)
    vn2 = jnp.sum(y * y, axis=-1, keepdims=True)
    vn = jnp.sqrt(vn2)
    arg = jnp.sqrt(cv * sf * vn2 / 2.0)
    second = (1.0 / jnp.sqrt(cv)) * jnp.tanh(arg) * y / vn
    ip = jnp.sum(rb * second, axis=-1, keepdims=True)
    sn2 = jnp.sum(second * second, axis=-1, keepdims=True)
    num = (1.0 + 2.0 * cv * ip + cv * sn2) * rb + (1.0 - cv * xn2) * second
    den = 1.0 + 2.0 * cv * ip + cv * cv * xn2 * sn2
    return num / den


def _combine_kernel(yg_ref, ref_ref, w8_ref, c_ref, out_ref):
    cv = c_ref[0, 0]
    rb = ref_ref[...]
    acc = jnp.zeros((BTC, N_EMBD), dtype=jnp.float32)
    for k in range(NPK):
        z = _expmap(rb, yg_ref[k], cv)
        acc = acc + w8_ref[:, k:k + 1] * z
    out_ref[...] = acc


def kernel(x, reference_point, c, Wg, bg, Wfc, bfc, Wproj, bproj):
    b, t, ch = x.shape
    x_flat = x.reshape(b * t, ch)
    ref_flat = reference_point.reshape(b * t, ch)

    wgt = jnp.zeros((N_EMBD, 128), dtype=jnp.float32).at[:, : N_EXPERTS - 1].set(Wg.T)
    bgp = jnp.full((1, 128), -1e30, dtype=jnp.float32).at[0, : N_EXPERTS - 1].set(bg)

    rw, idx8, w8 = pl.pallas_call(
        _gate_kernel,
        grid=(T // BTG,),
        in_specs=[
            pl.BlockSpec((BTG, N_EMBD), lambda i: (i, 0)),
            pl.BlockSpec((N_EMBD, 128), lambda i: (0, 0)),
            pl.BlockSpec((1, 128), lambda i: (0, 0)),
        ],
        out_specs=[
            pl.BlockSpec((BTG, N_EXPERTS), lambda i: (i, 0)),
            pl.BlockSpec((BTG, NPK), lambda i: (i, 0)),
            pl.BlockSpec((BTG, NPK), lambda i: (i, 0)),
        ],
        out_shape=[
            jax.ShapeDtypeStruct((T, N_EXPERTS), jnp.float32),
            jax.ShapeDtypeStruct((T, NPK), jnp.int32),
            jax.ShapeDtypeStruct((T, NPK), jnp.float32),
        ],
    )(x_flat, wgt, bgp)

    cnt = jax.nn.one_hot(idx8, N_EXPERTS, dtype=jnp.int32).sum(axis=1)
    ranks = jnp.cumsum(cnt, axis=0) - cnt
    counts = cnt.sum(axis=0)
    nb = (counts + BT - 1) // BT
    csum_nb = jnp.cumsum(nb)
    po = (csum_nb - nb) * BT
    nblocks = csum_nb[-1].reshape(1)
    pos_full = po[None, :] + ranks
    pos8 = jnp.take_along_axis(pos_full, idx8, axis=1)
    tokens8 = jnp.broadcast_to(jnp.arange(T, dtype=jnp.int32)[:, None], (T, NPK))
    sorted_tokens = jnp.zeros((PADTOT,), jnp.int32).at[pos8.reshape(-1)].set(
        tokens8.reshape(-1))
    block_expert = jnp.minimum(
        jnp.searchsorted(csum_nb, jnp.arange(NBLK, dtype=jnp.int32),
                         side="right"),
        N_EXPERTS - 1).astype(jnp.int32)

    xg_chunk, xg_nch = 128, PADTOT // (_NW * 128)
    x_sorted = _sc_gather(x_flat, sorted_tokens.reshape(_NW, xg_nch, xg_chunk),
                          xg_chunk, xg_nch, PADTOT)

    y_sorted = pl.pallas_call(
        _mlp_kernel,
        grid_spec=pltpu.PrefetchScalarGridSpec(
            num_scalar_prefetch=2,
            grid=(NBLK,),
            in_specs=[
                pl.BlockSpec((BT, N_EMBD), lambda bi, be, nbl: (bi, 0)),
                pl.BlockSpec((1, HIDDEN, N_EMBD),
                             lambda bi, be, nbl: (be[bi], 0, 0)),
                pl.BlockSpec((1, 1, HIDDEN), lambda bi, be, nbl: (be[bi], 0, 0)),
                pl.BlockSpec((1, N_EMBD, HIDDEN),
                             lambda bi, be, nbl: (be[bi], 0, 0)),
                pl.BlockSpec((1, 1, N_EMBD), lambda bi, be, nbl: (be[bi], 0, 0)),
            ],
            out_specs=pl.BlockSpec((BT, N_EMBD), lambda bi, be, nbl: (bi, 0)),
        ),
        out_shape=jax.ShapeDtypeStruct((PADTOT, N_EMBD), jnp.float32),
    )(block_expert, nblocks, x_sorted, Wfc.astype(jnp.bfloat16),
      bfc.reshape(N_EXPERTS, 1, HIDDEN), Wproj.astype(jnp.bfloat16),
      bproj.reshape(N_EXPERTS, 1, N_EMBD))

    idx_z = pos8.T.reshape(-1)
    zg_chunk, zg_nch = 128, (NPK * T) // (_NW * 128)
    yg = _sc_gather(y_sorted, idx_z.reshape(_NW, zg_nch, zg_chunk),
                    zg_chunk, zg_nch, NPK * T)

    out = pl.pallas_call(
        _combine_kernel,
        grid=(T // BTC,),
        in_specs=[
            pl.BlockSpec((NPK, BTC, N_EMBD), lambda i: (0, i, 0)),
            pl.BlockSpec((BTC, N_EMBD), lambda i: (i, 0)),
            pl.BlockSpec((BTC, NPK), lambda i: (i, 0)),
            pl.BlockSpec((1, 1), lambda i: (0, 0)),
        ],
        out_specs=pl.BlockSpec((BTC, N_EMBD), lambda i: (i, 0)),
        out_shape=jax.ShapeDtypeStruct((T, N_EMBD), jnp.float32),
    )(yg.reshape(NPK, T, N_EMBD), ref_flat, w8, c.reshape(1, 1))

    return (out.reshape(b, t, ch), rw)


# speedup vs baseline: 1.9895x; 1.9895x over previous
import functools

import jax
import jax.numpy as jnp
from jax import lax
from jax.experimental import pallas as pl
from jax.experimental.pallas import tpu as pltpu
from jax.experimental.pallas import tpu_sc as plsc

N_EXPERTS = 32
N_EMBD = 256
HIDDEN = 4 * N_EMBD
TOPK = 7
NPK = TOPK + 1
T = 2048

BTG = 256
BT = 128
BTC = 256
NBLK = 160
PADTOT = NBLK * BT

_NC, _NS = 2, 16
_NW = _NC * _NS

_INV_SQRT2 = 0.7071067811865476


def _gate_kernel(x_ref, wgt_ref, bgp_ref, rw_ref, idx_ref, w8_ref):
    xb = x_ref[...]
    logits = jnp.dot(xb, wgt_ref[...], preferred_element_type=jnp.float32)
    logits = logits + bgp_ref[...]
    m = jnp.max(logits, axis=1, keepdims=True)
    ex = jnp.exp(logits - m)
    z = jnp.sum(ex, axis=1, keepdims=True)
    p = ex / z
    lane = jax.lax.broadcasted_iota(jnp.int32, (BTG, N_EXPERTS), 1)
    valid = lane < (N_EXPERTS - 1)
    p = jnp.where(valid, p, -1.0)
    pcur = p
    selmask = jnp.zeros((BTG, N_EXPERTS), dtype=jnp.bool_)
    vals, idxs = [], []
    for _ in range(TOPK):
        mj = jnp.max(pcur, axis=1, keepdims=True)
        ismax = pcur == mj
        selidx = jnp.min(jnp.where(ismax, lane, N_EXPERTS - 1), axis=1,
                         keepdims=True)
        mask_j = lane == selidx
        selmask = jnp.logical_or(selmask, mask_j)
        vals.append(mj)
        idxs.append(selidx)
        pcur = jnp.where(mask_j, -1.0, pcur)
    psel = jnp.where(selmask, p, 0.0)
    s = jnp.sum(psel, axis=1, keepdims=True)
    scale = (TOPK / (TOPK + 1.0)) / s
    col0 = jnp.full((BTG, 1), 1.0 / (TOPK + 1.0), dtype=jnp.float32)
    rw_ref[...] = jnp.concatenate([col0, (psel * scale)[:, : N_EXPERTS - 1]],
                                  axis=1)
    idx_ref[...] = jnp.concatenate(
        [jnp.zeros((BTG, 1), jnp.int32)] + [i + 1 for i in idxs], axis=1)
    w8_ref[...] = jnp.concatenate([col0] + [v * scale for v in vals], axis=1)


def _sc_gather(table, idx3, chunk, nchunks, out_rows):
    d = table.shape[1]
    bpw = nchunks * chunk
    mesh = plsc.VectorSubcoreMesh(core_axis_name="c", subcore_axis_name="s")

    @functools.partial(
        pl.kernel,
        mesh=mesh,
        out_type=jax.ShapeDtypeStruct((out_rows, d), jnp.float32),
        scratch_types=[
            pltpu.VMEM((nchunks, chunk), jnp.int32),
            pltpu.VMEM((chunk, d), jnp.float32),
            pltpu.SemaphoreType.DMA,
        ],
    )
    def gather_k(table_hbm, idx_hbm, out_hbm, idx_v, rows_v, sem):
        wid = lax.axis_index("s") * _NC + lax.axis_index("c")
        base = wid * bpw
        pltpu.sync_copy(idx_hbm.at[wid], idx_v)
        for ci in range(nchunks):
            pltpu.async_copy(table_hbm.at[idx_v.at[ci]], rows_v, sem).wait()
            pltpu.sync_copy(rows_v, out_hbm.at[pl.ds(base + ci * chunk, chunk)])

    return gather_k(table, idx3)


def _sc_scatter_rows(table, idx3, out_rows):
    d = table.shape[1]
    tpw = table.shape[0] // _NW
    mesh = plsc.VectorSubcoreMesh(core_axis_name="c", subcore_axis_name="s")

    @functools.partial(
        pl.kernel,
        mesh=mesh,
        out_type=jax.ShapeDtypeStruct((out_rows, d), jnp.float32),
        scratch_types=[
            pltpu.VMEM((NPK, tpw), jnp.int32),
            pltpu.VMEM((tpw, d), jnp.float32),
            pltpu.SemaphoreType.DMA,
        ],
    )
    def scatter_k(table_hbm, idx_hbm, out_hbm, idx_v, rows_v, sem):
        wid = lax.axis_index("s") * _NC + lax.axis_index("c")
        pltpu.sync_copy(idx_hbm.at[wid], idx_v)
        pltpu.sync_copy(table_hbm.at[pl.ds(wid * tpw, tpw)], rows_v)
        copies = [
            pltpu.async_copy(rows_v, out_hbm.at[idx_v.at[k]], sem)
            for k in range(NPK)
        ]
        for cp in copies:
            cp.wait()

    return scatter_k(table, idx3)


def _mlp_kernel(be_ref, nb_ref, x_ref, wfc_ref, bfc_ref, wproj_ref, bproj_ref,
                y_ref):
    b = pl.program_id(0)

    @pl.when(b < nb_ref[0])
    def _():
        xb = x_ref[...].astype(jnp.bfloat16)
        h = jax.lax.dot_general(xb, wfc_ref[0], (((1,), (1,)), ((), ())),
                                preferred_element_type=jnp.float32)
        h = h + bfc_ref[0]
        h = 0.5 * h * (1.0 + jax.lax.erf(h * _INV_SQRT2))
        y = jax.lax.dot_general(h.astype(jnp.bfloat16), wproj_ref[0],
                                (((1,), (1,)), ((), ())),
                                preferred_element_type=jnp.float32)
        y_ref[...] = y + bproj_ref[0]


def _expmap(rb, y, cv):
    xn2 = jnp.sum(rb * rb, axis=-1, keepdims=True)
    sf = 2.0 / (1.0 + cv * xn2)
    vn2 = jnp.sum(y * y, axis=-1, keepdims=True)
    vn = jnp.sqrt(vn2)
    arg = jnp.sqrt(cv * sf * vn2 / 2.0)
    second = (1.0 / jnp.sqrt(cv)) * jnp.tanh(arg) * y / vn
    ip = jnp.sum(rb * second, axis=-1, keepdims=True)
    sn2 = jnp.sum(second * second, axis=-1, keepdims=True)
    num = (1.0 + 2.0 * cv * ip + cv * sn2) * rb + (1.0 - cv * xn2) * second
    den = 1.0 + 2.0 * cv * ip + cv * cv * xn2 * sn2
    return num / den


def _combine_kernel(yg_ref, ref_ref, w8_ref, c_ref, out_ref):
    cv = c_ref[0, 0]
    rb = ref_ref[...]
    acc = jnp.zeros((BTC, N_EMBD), dtype=jnp.float32)
    for k in range(NPK):
        z = _expmap(rb, yg_ref[k], cv)
        acc = acc + w8_ref[:, k:k + 1] * z
    out_ref[...] = acc


def kernel(x, reference_point, c, Wg, bg, Wfc, bfc, Wproj, bproj):
    b, t, ch = x.shape
    x_flat = x.reshape(b * t, ch)
    ref_flat = reference_point.reshape(b * t, ch)

    wgt = jnp.zeros((N_EMBD, N_EXPERTS), dtype=jnp.float32).at[:, : N_EXPERTS - 1].set(Wg.T)
    bgp = jnp.full((1, N_EXPERTS), -1e30, dtype=jnp.float32).at[0, : N_EXPERTS - 1].set(bg)

    rw, idx8, w8 = pl.pallas_call(
        _gate_kernel,
        grid=(T // BTG,),
        in_specs=[
            pl.BlockSpec((BTG, N_EMBD), lambda i: (i, 0)),
            pl.BlockSpec((N_EMBD, N_EXPERTS), lambda i: (0, 0)),
            pl.BlockSpec((1, N_EXPERTS), lambda i: (0, 0)),
        ],
        out_specs=[
            pl.BlockSpec((BTG, N_EXPERTS), lambda i: (i, 0)),
            pl.BlockSpec((BTG, NPK), lambda i: (i, 0)),
            pl.BlockSpec((BTG, NPK), lambda i: (i, 0)),
        ],
        out_shape=[
            jax.ShapeDtypeStruct((T, N_EXPERTS), jnp.float32),
            jax.ShapeDtypeStruct((T, NPK), jnp.int32),
            jax.ShapeDtypeStruct((T, NPK), jnp.float32),
        ],
    )(x_flat, wgt, bgp)

    cnt = jax.nn.one_hot(idx8, N_EXPERTS, dtype=jnp.int32).sum(axis=1)
    ranks = jnp.cumsum(cnt, axis=0) - cnt
    counts = cnt.sum(axis=0)
    nb = (counts + BT - 1) // BT
    csum_nb = jnp.cumsum(nb)
    po = (csum_nb - nb) * BT
    nblocks = csum_nb[-1].reshape(1)
    pos_full = po[None, :] + ranks
    pos8 = jnp.take_along_axis(pos_full, idx8, axis=1)
    block_expert = jnp.minimum(
        jnp.sum(jnp.arange(NBLK, dtype=jnp.int32)[:, None] >= csum_nb[None, :],
                axis=1),
        N_EXPERTS - 1).astype(jnp.int32)

    idx_s = pos8.reshape(_NW, T // _NW, NPK).transpose(0, 2, 1)
    x_sorted = _sc_scatter_rows(x_flat, idx_s, PADTOT)

    y_sorted = pl.pallas_call(
        _mlp_kernel,
        grid_spec=pltpu.PrefetchScalarGridSpec(
            num_scalar_prefetch=2,
            grid=(NBLK,),
            in_specs=[
                pl.BlockSpec((BT, N_EMBD), lambda bi, be, nbl: (bi, 0)),
                pl.BlockSpec((1, HIDDEN, N_EMBD),
                             lambda bi, be, nbl: (be[bi], 0, 0)),
                pl.BlockSpec((1, 1, HIDDEN), lambda bi, be, nbl: (be[bi], 0, 0)),
                pl.BlockSpec((1, N_EMBD, HIDDEN),
                             lambda bi, be, nbl: (be[bi], 0, 0)),
                pl.BlockSpec((1, 1, N_EMBD), lambda bi, be, nbl: (be[bi], 0, 0)),
            ],
            out_specs=pl.BlockSpec((BT, N_EMBD), lambda bi, be, nbl: (bi, 0)),
        ),
        out_shape=jax.ShapeDtypeStruct((PADTOT, N_EMBD), jnp.float32),
    )(block_expert, nblocks, x_sorted, Wfc.astype(jnp.bfloat16),
      bfc.reshape(N_EXPERTS, 1, HIDDEN), Wproj.astype(jnp.bfloat16),
      bproj.reshape(N_EXPERTS, 1, N_EMBD))

    idx_z = pos8.T.reshape(-1)
    zg_chunk, zg_nch = 128, (NPK * T) // (_NW * 128)
    yg = _sc_gather(y_sorted, idx_z.reshape(_NW, zg_nch, zg_chunk),
                    zg_chunk, zg_nch, NPK * T)

    out = pl.pallas_call(
        _combine_kernel,
        grid=(T // BTC,),
        in_specs=[
            pl.BlockSpec((NPK, BTC, N_EMBD), lambda i: (0, i, 0)),
            pl.BlockSpec((BTC, N_EMBD), lambda i: (i, 0)),
            pl.BlockSpec((BTC, NPK), lambda i: (i, 0)),
            pl.BlockSpec((1, 1), lambda i: (0, 0)),
        ],
        out_specs=pl.BlockSpec((BTC, N_EMBD), lambda i: (i, 0)),
        out_shape=jax.ShapeDtypeStruct((T, N_EMBD), jnp.float32),
    )(yg.reshape(NPK, T, N_EMBD), ref_flat, w8, c.reshape(1, 1))

    return (out.reshape(b, t, ch), rw)


# speedup vs baseline: 2.3883x; 1.2005x over previous
import functools

import jax
import jax.numpy as jnp
from jax import lax
from jax.experimental import pallas as pl
from jax.experimental.pallas import tpu as pltpu
from jax.experimental.pallas import tpu_sc as plsc

N_EXPERTS = 32
N_EMBD = 256
HIDDEN = 4 * N_EMBD
TOPK = 7
NPK = TOPK + 1
T = 2048

BTG = 256
BT = 256
BTC = 256
NBLK = 96
PADTOT = NBLK * BT

_NC, _NS = 2, 16
_NW = _NC * _NS

_INV_SQRT2 = 0.7071067811865476


def _gate_kernel(x_ref, wgt_ref, bgp_ref, rw_ref, idx_ref, w8_ref):
    xb = x_ref[...]
    logits = jnp.dot(xb, wgt_ref[...], preferred_element_type=jnp.float32)
    logits = logits + bgp_ref[...]
    m = jnp.max(logits, axis=1, keepdims=True)
    ex = jnp.exp(logits - m)
    z = jnp.sum(ex, axis=1, keepdims=True)
    p = ex / z
    lane = jax.lax.broadcasted_iota(jnp.int32, (BTG, N_EXPERTS), 1)
    valid = lane < (N_EXPERTS - 1)
    p = jnp.where(valid, p, -1.0)
    pcur = p
    selmask = jnp.zeros((BTG, N_EXPERTS), dtype=jnp.bool_)
    vals, idxs = [], []
    for _ in range(TOPK):
        mj = jnp.max(pcur, axis=1, keepdims=True)
        ismax = pcur == mj
        selidx = jnp.min(jnp.where(ismax, lane, N_EXPERTS - 1), axis=1,
                         keepdims=True)
        mask_j = lane == selidx
        selmask = jnp.logical_or(selmask, mask_j)
        vals.append(mj)
        idxs.append(selidx)
        pcur = jnp.where(mask_j, -1.0, pcur)
    psel = jnp.where(selmask, p, 0.0)
    s = jnp.sum(psel, axis=1, keepdims=True)
    scale = (TOPK / (TOPK + 1.0)) / s
    col0 = jnp.full((BTG, 1), 1.0 / (TOPK + 1.0), dtype=jnp.float32)
    rw_ref[...] = jnp.concatenate([col0, (psel * scale)[:, : N_EXPERTS - 1]],
                                  axis=1)
    idx_ref[...] = jnp.concatenate(
        [jnp.zeros((BTG, 1), jnp.int32)] + [i + 1 for i in idxs], axis=1)
    w8_ref[...] = jnp.concatenate([col0] + [v * scale for v in vals], axis=1)


def _sc_gather(table, idx3, chunk, nchunks, out_rows):
    d = table.shape[1]
    bpw = nchunks * chunk
    mesh = plsc.VectorSubcoreMesh(core_axis_name="c", subcore_axis_name="s")

    @functools.partial(
        pl.kernel,
        mesh=mesh,
        out_type=jax.ShapeDtypeStruct((out_rows, d), jnp.float32),
        scratch_types=[
            pltpu.VMEM((nchunks, chunk), jnp.int32),
            pltpu.VMEM((chunk, d), jnp.float32),
            pltpu.SemaphoreType.DMA,
        ],
    )
    def gather_k(table_hbm, idx_hbm, out_hbm, idx_v, rows_v, sem):
        wid = lax.axis_index("s") * _NC + lax.axis_index("c")
        base = wid * bpw
        pltpu.sync_copy(idx_hbm.at[wid], idx_v)
        for ci in range(nchunks):
            pltpu.async_copy(table_hbm.at[idx_v.at[ci]], rows_v, sem).wait()
            pltpu.sync_copy(rows_v, out_hbm.at[pl.ds(base + ci * chunk, chunk)])

    return gather_k(table, idx3)


def _sc_scatter_rows(table, idx3, out_rows):
    d = table.shape[1]
    tpw = table.shape[0] // _NW
    mesh = plsc.VectorSubcoreMesh(core_axis_name="c", subcore_axis_name="s")

    @functools.partial(
        pl.kernel,
        mesh=mesh,
        out_type=jax.ShapeDtypeStruct((out_rows, d), jnp.float32),
        scratch_types=[
            pltpu.VMEM((NPK, tpw), jnp.int32),
            pltpu.VMEM((tpw, d), jnp.float32),
            pltpu.SemaphoreType.DMA,
        ],
    )
    def scatter_k(table_hbm, idx_hbm, out_hbm, idx_v, rows_v, sem):
        wid = lax.axis_index("s") * _NC + lax.axis_index("c")
        pltpu.sync_copy(idx_hbm.at[wid], idx_v)
        pltpu.sync_copy(table_hbm.at[pl.ds(wid * tpw, tpw)], rows_v)
        copies = [
            pltpu.async_copy(rows_v, out_hbm.at[idx_v.at[k]], sem)
            for k in range(NPK)
        ]
        for cp in copies:
            cp.wait()

    return scatter_k(table, idx3)


def _mlp_kernel(be_ref, nb_ref, x_ref, wfc_ref, bfc_ref, wproj_ref, bproj_ref,
                y_ref):
    b = pl.program_id(0)

    @pl.when(b < nb_ref[0])
    def _():
        xb = x_ref[...].astype(jnp.bfloat16)
        h = jax.lax.dot_general(xb, wfc_ref[0], (((1,), (1,)), ((), ())),
                                preferred_element_type=jnp.float32)
        h = h + bfc_ref[0]
        h = 0.5 * h * (1.0 + jax.lax.erf(h * _INV_SQRT2))
        y = jax.lax.dot_general(h.astype(jnp.bfloat16), wproj_ref[0],
                                (((1,), (1,)), ((), ())),
                                preferred_element_type=jnp.float32)
        y_ref[...] = y + bproj_ref[0]


def _expmap(rb, y, cv):
    xn2 = jnp.sum(rb * rb, axis=-1, keepdims=True)
    sf = 2.0 / (1.0 + cv * xn2)
    vn2 = jnp.sum(y * y, axis=-1, keepdims=True)
    vn = jnp.sqrt(vn2)
    arg = jnp.sqrt(cv * sf * vn2 / 2.0)
    second = (1.0 / jnp.sqrt(cv)) * jnp.tanh(arg) * y / vn
    ip = jnp.sum(rb * second, axis=-1, keepdims=True)
    sn2 = jnp.sum(second * second, axis=-1, keepdims=True)
    num = (1.0 + 2.0 * cv * ip + cv * sn2) * rb + (1.0 - cv * xn2) * second
    den = 1.0 + 2.0 * cv * ip + cv * cv * xn2 * sn2
    return num / den


def _combine_kernel(yg_ref, ref_ref, w8_ref, c_ref, out_ref):
    cv = c_ref[0, 0]
    rb = ref_ref[...]
    acc = jnp.zeros((BTC, N_EMBD), dtype=jnp.float32)
    for k in range(NPK):
        z = _expmap(rb, yg_ref[k], cv)
        acc = acc + w8_ref[:, k:k + 1] * z
    out_ref[...] = acc


def kernel(x, reference_point, c, Wg, bg, Wfc, bfc, Wproj, bproj):
    b, t, ch = x.shape
    x_flat = x.reshape(b * t, ch)
    ref_flat = reference_point.reshape(b * t, ch)

    wgt = jnp.zeros((N_EMBD, N_EXPERTS), dtype=jnp.float32).at[:, : N_EXPERTS - 1].set(Wg.T)
    bgp = jnp.full((1, N_EXPERTS), -1e30, dtype=jnp.float32).at[0, : N_EXPERTS - 1].set(bg)

    rw, idx8, w8 = pl.pallas_call(
        _gate_kernel,
        grid=(T // BTG,),
        in_specs=[
            pl.BlockSpec((BTG, N_EMBD), lambda i: (i, 0)),
            pl.BlockSpec((N_EMBD, N_EXPERTS), lambda i: (0, 0)),
            pl.BlockSpec((1, N_EXPERTS), lambda i: (0, 0)),
        ],
        out_specs=[
            pl.BlockSpec((BTG, N_EXPERTS), lambda i: (i, 0)),
            pl.BlockSpec((BTG, NPK), lambda i: (i, 0)),
            pl.BlockSpec((BTG, NPK), lambda i: (i, 0)),
        ],
        out_shape=[
            jax.ShapeDtypeStruct((T, N_EXPERTS), jnp.float32),
            jax.ShapeDtypeStruct((T, NPK), jnp.int32),
            jax.ShapeDtypeStruct((T, NPK), jnp.float32),
        ],
    )(x_flat, wgt, bgp)

    cnt = jax.nn.one_hot(idx8, N_EXPERTS, dtype=jnp.int32).sum(axis=1)
    ranks = jnp.cumsum(cnt, axis=0) - cnt
    counts = cnt.sum(axis=0)
    nb = (counts + BT - 1) // BT
    csum_nb = jnp.cumsum(nb)
    po = (csum_nb - nb) * BT
    nblocks = csum_nb[-1].reshape(1)
    pos_full = po[None, :] + ranks
    pos8 = jnp.take_along_axis(pos_full, idx8, axis=1)
    block_expert = jnp.minimum(
        jnp.sum(jnp.arange(NBLK, dtype=jnp.int32)[:, None] >= csum_nb[None, :],
                axis=1),
        N_EXPERTS - 1).astype(jnp.int32)

    idx_s = pos8.reshape(_NW, T // _NW, NPK).transpose(0, 2, 1)
    x_sorted = _sc_scatter_rows(x_flat, idx_s, PADTOT)

    y_sorted = pl.pallas_call(
        _mlp_kernel,
        grid_spec=pltpu.PrefetchScalarGridSpec(
            num_scalar_prefetch=2,
            grid=(NBLK,),
            in_specs=[
                pl.BlockSpec((BT, N_EMBD), lambda bi, be, nbl: (bi, 0)),
                pl.BlockSpec((1, HIDDEN, N_EMBD),
                             lambda bi, be, nbl: (be[bi], 0, 0)),
                pl.BlockSpec((1, 1, HIDDEN), lambda bi, be, nbl: (be[bi], 0, 0)),
                pl.BlockSpec((1, N_EMBD, HIDDEN),
                             lambda bi, be, nbl: (be[bi], 0, 0)),
                pl.BlockSpec((1, 1, N_EMBD), lambda bi, be, nbl: (be[bi], 0, 0)),
            ],
            out_specs=pl.BlockSpec((BT, N_EMBD), lambda bi, be, nbl: (bi, 0)),
        ),
        out_shape=jax.ShapeDtypeStruct((PADTOT, N_EMBD), jnp.float32),
    )(block_expert, nblocks, x_sorted, Wfc.astype(jnp.bfloat16),
      bfc.reshape(N_EXPERTS, 1, HIDDEN), Wproj.astype(jnp.bfloat16),
      bproj.reshape(N_EXPERTS, 1, N_EMBD))

    idx_z = pos8.T.reshape(-1)
    zg_chunk, zg_nch = 128, (NPK * T) // (_NW * 128)
    yg = _sc_gather(y_sorted, idx_z.reshape(_NW, zg_nch, zg_chunk),
                    zg_chunk, zg_nch, NPK * T)

    out = pl.pallas_call(
        _combine_kernel,
        grid=(T // BTC,),
        in_specs=[
            pl.BlockSpec((NPK, BTC, N_EMBD), lambda i: (0, i, 0)),
            pl.BlockSpec((BTC, N_EMBD), lambda i: (i, 0)),
            pl.BlockSpec((BTC, NPK), lambda i: (i, 0)),
            pl.BlockSpec((1, 1), lambda i: (0, 0)),
        ],
        out_specs=pl.BlockSpec((BTC, N_EMBD), lambda i: (i, 0)),
        out_shape=jax.ShapeDtypeStruct((T, N_EMBD), jnp.float32),
    )(yg.reshape(NPK, T, N_EMBD), ref_flat, w8, c.reshape(1, 1))

    return (out.reshape(b, t, ch), rw)


# speedup vs baseline: 3.0634x; 1.2826x over previous
import functools

import jax
import jax.numpy as jnp
from jax import lax
from jax.experimental import pallas as pl
from jax.experimental.pallas import tpu as pltpu
from jax.experimental.pallas import tpu_sc as plsc

N_EXPERTS = 32
N_EMBD = 256
HIDDEN = 4 * N_EMBD
TOPK = 7
NPK = TOPK + 1
T = 2048

BTG = 256
BT = 512
BTC = 256
NBLK = 64
PADTOT = NBLK * BT

_NC, _NS = 2, 16
_NW = _NC * _NS

_INV_SQRT2 = 0.7071067811865476


def _gate_kernel(x_ref, wgt_ref, bgp_ref, rw_ref, idx_ref, w8_ref):
    xb = x_ref[...]
    logits = jnp.dot(xb, wgt_ref[...], preferred_element_type=jnp.float32)
    logits = logits + bgp_ref[...]
    m = jnp.max(logits, axis=1, keepdims=True)
    ex = jnp.exp(logits - m)
    z = jnp.sum(ex, axis=1, keepdims=True)
    p = ex / z
    lane = jax.lax.broadcasted_iota(jnp.int32, (BTG, N_EXPERTS), 1)
    valid = lane < (N_EXPERTS - 1)
    p = jnp.where(valid, p, -1.0)
    pcur = p
    selmask = jnp.zeros((BTG, N_EXPERTS), dtype=jnp.bool_)
    vals, idxs = [], []
    for _ in range(TOPK):
        mj = jnp.max(pcur, axis=1, keepdims=True)
        ismax = pcur == mj
        selidx = jnp.min(jnp.where(ismax, lane, N_EXPERTS - 1), axis=1,
                         keepdims=True)
        mask_j = lane == selidx
        selmask = jnp.logical_or(selmask, mask_j)
        vals.append(mj)
        idxs.append(selidx)
        pcur = jnp.where(mask_j, -1.0, pcur)
    psel = jnp.where(selmask, p, 0.0)
    s = jnp.sum(psel, axis=1, keepdims=True)
    scale = (TOPK / (TOPK + 1.0)) / s
    col0 = jnp.full((BTG, 1), 1.0 / (TOPK + 1.0), dtype=jnp.float32)
    rw_ref[...] = jnp.concatenate([col0, (psel * scale)[:, : N_EXPERTS - 1]],
                                  axis=1)
    idx_ref[...] = jnp.concatenate(
        [jnp.zeros((BTG, 1), jnp.int32)] + [i + 1 for i in idxs], axis=1)
    w8_ref[...] = jnp.concatenate([col0] + [v * scale for v in vals], axis=1)


def _sc_gather(table, idx3, chunk, nchunks, out_rows):
    d = table.shape[1]
    bpw = nchunks * chunk
    mesh = plsc.VectorSubcoreMesh(core_axis_name="c", subcore_axis_name="s")

    @functools.partial(
        pl.kernel,
        mesh=mesh,
        out_type=jax.ShapeDtypeStruct((out_rows, d), jnp.float32),
        scratch_types=[
            pltpu.VMEM((nchunks, chunk), jnp.int32),
            pltpu.VMEM((chunk, d), jnp.float32),
            pltpu.SemaphoreType.DMA,
        ],
    )
    def gather_k(table_hbm, idx_hbm, out_hbm, idx_v, rows_v, sem):
        wid = lax.axis_index("s") * _NC + lax.axis_index("c")
        base = wid * bpw
        pltpu.sync_copy(idx_hbm.at[wid], idx_v)
        for ci in range(nchunks):
            pltpu.async_copy(table_hbm.at[idx_v.at[ci]], rows_v, sem).wait()
            pltpu.sync_copy(rows_v, out_hbm.at[pl.ds(base + ci * chunk, chunk)])

    return gather_k(table, idx3)


def _sc_scatter_rows(table, idx3, out_rows):
    d = table.shape[1]
    tpw = table.shape[0] // _NW
    mesh = plsc.VectorSubcoreMesh(core_axis_name="c", subcore_axis_name="s")

    @functools.partial(
        pl.kernel,
        mesh=mesh,
        out_type=jax.ShapeDtypeStruct((out_rows, d), jnp.float32),
        scratch_types=[
            pltpu.VMEM((NPK, tpw), jnp.int32),
            pltpu.VMEM((tpw, d), jnp.float32),
            pltpu.SemaphoreType.DMA,
        ],
    )
    def scatter_k(table_hbm, idx_hbm, out_hbm, idx_v, rows_v, sem):
        wid = lax.axis_index("s") * _NC + lax.axis_index("c")
        pltpu.sync_copy(idx_hbm.at[wid], idx_v)
        pltpu.sync_copy(table_hbm.at[pl.ds(wid * tpw, tpw)], rows_v)
        copies = [
            pltpu.async_copy(rows_v, out_hbm.at[idx_v.at[k]], sem)
            for k in range(NPK)
        ]
        for cp in copies:
            cp.wait()

    return scatter_k(table, idx3)


def _mlp_kernel(be_ref, nb_ref, x_ref, wfc_ref, bfc_ref, wproj_ref, bproj_ref,
                y_ref):
    b = pl.program_id(0)

    @pl.when(b < nb_ref[0])
    def _():
        xb = x_ref[...].astype(jnp.bfloat16)
        h = jax.lax.dot_general(xb, wfc_ref[0].astype(jnp.bfloat16),
                                (((1,), (1,)), ((), ())),
                                preferred_element_type=jnp.float32)
        h = h + bfc_ref[0]
        h = 0.5 * h * (1.0 + jax.lax.erf(h * _INV_SQRT2))
        y = jax.lax.dot_general(h.astype(jnp.bfloat16),
                                wproj_ref[0].astype(jnp.bfloat16),
                                (((1,), (1,)), ((), ())),
                                preferred_element_type=jnp.float32)
        y_ref[...] = y + bproj_ref[0]


def _expmap(rb, y, cv):
    xn2 = jnp.sum(rb * rb, axis=-1, keepdims=True)
    sf = 2.0 / (1.0 + cv * xn2)
    vn2 = jnp.sum(y * y, axis=-1, keepdims=True)
    vn = jnp.sqrt(vn2)
    arg = jnp.sqrt(cv * sf * vn2 / 2.0)
    second = (1.0 / jnp.sqrt(cv)) * jnp.tanh(arg) * y / vn
    ip = jnp.sum(rb * second, axis=-1, keepdims=True)
    sn2 = jnp.sum(second * second, axis=-1, keepdims=True)
    num = (1.0 + 2.0 * cv * ip + cv * sn2) * rb + (1.0 - cv * xn2) * second
    den = 1.0 + 2.0 * cv * ip + cv * cv * xn2 * sn2
    return num / den


def _combine_kernel(yg_ref, ref_ref, w8_ref, c_ref, out_ref):
    cv = c_ref[0, 0]
    rb = ref_ref[...]
    acc = jnp.zeros((BTC, N_EMBD), dtype=jnp.float32)
    for k in range(NPK):
        z = _expmap(rb, yg_ref[k], cv)
        acc = acc + w8_ref[:, k:k + 1] * z
    out_ref[...] = acc


def kernel(x, reference_point, c, Wg, bg, Wfc, bfc, Wproj, bproj):
    b, t, ch = x.shape
    x_flat = x.reshape(b * t, ch)
    ref_flat = reference_point.reshape(b * t, ch)

    wgt = jnp.zeros((N_EMBD, N_EXPERTS), dtype=jnp.float32).at[:, : N_EXPERTS - 1].set(Wg.T)
    bgp = jnp.full((1, N_EXPERTS), -1e30, dtype=jnp.float32).at[0, : N_EXPERTS - 1].set(bg)

    rw, idx8, w8 = pl.pallas_call(
        _gate_kernel,
        grid=(T // BTG,),
        in_specs=[
            pl.BlockSpec((BTG, N_EMBD), lambda i: (i, 0)),
            pl.BlockSpec((N_EMBD, N_EXPERTS), lambda i: (0, 0)),
            pl.BlockSpec((1, N_EXPERTS), lambda i: (0, 0)),
        ],
        out_specs=[
            pl.BlockSpec((BTG, N_EXPERTS), lambda i: (i, 0)),
            pl.BlockSpec((BTG, NPK), lambda i: (i, 0)),
            pl.BlockSpec((BTG, NPK), lambda i: (i, 0)),
        ],
        out_shape=[
            jax.ShapeDtypeStruct((T, N_EXPERTS), jnp.float32),
            jax.ShapeDtypeStruct((T, NPK), jnp.int32),
            jax.ShapeDtypeStruct((T, NPK), jnp.float32),
        ],
    )(x_flat, wgt, bgp)

    cnt = jax.nn.one_hot(idx8, N_EXPERTS, dtype=jnp.int32).sum(axis=1)
    ranks = jnp.cumsum(cnt, axis=0) - cnt
    counts = cnt.sum(axis=0)
    nb = (counts + BT - 1) // BT
    csum_nb = jnp.cumsum(nb)
    po = (csum_nb - nb) * BT
    nblocks = csum_nb[-1].reshape(1)
    pos_full = po[None, :] + ranks
    pos8 = jnp.take_along_axis(pos_full, idx8, axis=1)
    block_expert = jnp.minimum(
        jnp.sum(jnp.arange(NBLK, dtype=jnp.int32)[:, None] >= csum_nb[None, :],
                axis=1),
        N_EXPERTS - 1).astype(jnp.int32)

    idx_s = pos8.reshape(_NW, T // _NW, NPK).transpose(0, 2, 1)
    x_sorted = _sc_scatter_rows(x_flat, idx_s, PADTOT)

    y_sorted = pl.pallas_call(
        _mlp_kernel,
        grid_spec=pltpu.PrefetchScalarGridSpec(
            num_scalar_prefetch=2,
            grid=(NBLK,),
            in_specs=[
                pl.BlockSpec((BT, N_EMBD), lambda bi, be, nbl: (bi, 0)),
                pl.BlockSpec((1, HIDDEN, N_EMBD),
                             lambda bi, be, nbl: (be[bi], 0, 0)),
                pl.BlockSpec((1, 1, HIDDEN), lambda bi, be, nbl: (be[bi], 0, 0)),
                pl.BlockSpec((1, N_EMBD, HIDDEN),
                             lambda bi, be, nbl: (be[bi], 0, 0)),
                pl.BlockSpec((1, 1, N_EMBD), lambda bi, be, nbl: (be[bi], 0, 0)),
            ],
            out_specs=pl.BlockSpec((BT, N_EMBD), lambda bi, be, nbl: (bi, 0)),
        ),
        out_shape=jax.ShapeDtypeStruct((PADTOT, N_EMBD), jnp.float32),
    )(block_expert, nblocks, x_sorted, Wfc,
      bfc.reshape(N_EXPERTS, 1, HIDDEN), Wproj,
      bproj.reshape(N_EXPERTS, 1, N_EMBD))

    idx_z = pos8.T.reshape(-1)
    zg_chunk, zg_nch = 128, (NPK * T) // (_NW * 128)
    yg = _sc_gather(y_sorted, idx_z.reshape(_NW, zg_nch, zg_chunk),
                    zg_chunk, zg_nch, NPK * T)

    out = pl.pallas_call(
        _combine_kernel,
        grid=(T // BTC,),
        in_specs=[
            pl.BlockSpec((NPK, BTC, N_EMBD), lambda i: (0, i, 0)),
            pl.BlockSpec((BTC, N_EMBD), lambda i: (i, 0)),
            pl.BlockSpec((BTC, NPK), lambda i: (i, 0)),
            pl.BlockSpec((1, 1), lambda i: (0, 0)),
        ],
        out_specs=pl.BlockSpec((BTC, N_EMBD), lambda i: (i, 0)),
        out_shape=jax.ShapeDtypeStruct((T, N_EMBD), jnp.float32),
    )(yg.reshape(NPK, T, N_EMBD), ref_flat, w8, c.reshape(1, 1))

    return (out.reshape(b, t, ch), rw)


# speedup vs baseline: 3.1118x; 1.0158x over previous
import functools

import jax
import jax.numpy as jnp
from jax import lax
from jax.experimental import pallas as pl
from jax.experimental.pallas import tpu as pltpu
from jax.experimental.pallas import tpu_sc as plsc

N_EXPERTS = 32
N_EMBD = 256
HIDDEN = 4 * N_EMBD
TOPK = 7
NPK = TOPK + 1
T = 2048

BTG = 256
BT = 512
BTC = 256
NBLK = 64
PADTOT = NBLK * BT

_NC, _NS = 2, 16
_NW = _NC * _NS

_INV_SQRT2 = 0.7071067811865476


def _gate_kernel(x_ref, wgt_ref, bgp_ref, rw_ref, idx_ref, w8_ref):
    xb = x_ref[...]
    logits = jnp.dot(xb, wgt_ref[...], preferred_element_type=jnp.float32)
    logits = logits + bgp_ref[...]
    m = jnp.max(logits, axis=1, keepdims=True)
    ex = jnp.exp(logits - m)
    z = jnp.sum(ex, axis=1, keepdims=True)
    p = ex / z
    lane = jax.lax.broadcasted_iota(jnp.int32, (BTG, N_EXPERTS), 1)
    valid = lane < (N_EXPERTS - 1)
    p = jnp.where(valid, p, -1.0)
    pcur = p
    selmask = jnp.zeros((BTG, N_EXPERTS), dtype=jnp.bool_)
    vals, idxs = [], []
    for _ in range(TOPK):
        mj = jnp.max(pcur, axis=1, keepdims=True)
        ismax = pcur == mj
        selidx = jnp.min(jnp.where(ismax, lane, N_EXPERTS - 1), axis=1,
                         keepdims=True)
        mask_j = lane == selidx
        selmask = jnp.logical_or(selmask, mask_j)
        vals.append(mj)
        idxs.append(selidx)
        pcur = jnp.where(mask_j, -1.0, pcur)
    psel = jnp.where(selmask, p, 0.0)
    s = jnp.sum(psel, axis=1, keepdims=True)
    scale = (TOPK / (TOPK + 1.0)) / s
    col0 = jnp.full((BTG, 1), 1.0 / (TOPK + 1.0), dtype=jnp.float32)
    rw_ref[...] = jnp.concatenate([col0, (psel * scale)[:, : N_EXPERTS - 1]],
                                  axis=1)
    idx_ref[...] = jnp.concatenate(
        [jnp.zeros((BTG, 1), jnp.int32)] + [i + 1 for i in idxs], axis=1)
    w8_ref[...] = jnp.concatenate([col0] + [v * scale for v in vals], axis=1)


def _sc_gather(table, idx3, chunk, nchunks, out_rows):
    d = table.shape[1]
    bpw = nchunks * chunk
    mesh = plsc.VectorSubcoreMesh(core_axis_name="c", subcore_axis_name="s")

    @functools.partial(
        pl.kernel,
        mesh=mesh,
        out_type=jax.ShapeDtypeStruct((out_rows, d), jnp.float32),
        scratch_types=[
            pltpu.VMEM((nchunks, chunk), jnp.int32),
            pltpu.VMEM((chunk, d), jnp.float32),
            pltpu.SemaphoreType.DMA,
        ],
    )
    def gather_k(table_hbm, idx_hbm, out_hbm, idx_v, rows_v, sem):
        wid = lax.axis_index("s") * _NC + lax.axis_index("c")
        base = wid * bpw
        pltpu.sync_copy(idx_hbm.at[wid], idx_v)
        for ci in range(nchunks):
            pltpu.async_copy(table_hbm.at[idx_v.at[ci]], rows_v, sem).wait()
            pltpu.sync_copy(rows_v, out_hbm.at[pl.ds(base + ci * chunk, chunk)])

    return gather_k(table, idx3)


def _sc_scatter_rows(table, idx3, out_rows):
    d = table.shape[1]
    tpw = table.shape[0] // _NW
    mesh = plsc.VectorSubcoreMesh(core_axis_name="c", subcore_axis_name="s")

    @functools.partial(
        pl.kernel,
        mesh=mesh,
        out_type=jax.ShapeDtypeStruct((out_rows, d), jnp.float32),
        scratch_types=[
            pltpu.VMEM((NPK, tpw), jnp.int32),
            pltpu.VMEM((tpw, d), jnp.float32),
            pltpu.SemaphoreType.DMA,
        ],
    )
    def scatter_k(table_hbm, idx_hbm, out_hbm, idx_v, rows_v, sem):
        wid = lax.axis_index("s") * _NC + lax.axis_index("c")
        pltpu.sync_copy(idx_hbm.at[wid], idx_v)
        pltpu.sync_copy(table_hbm.at[pl.ds(wid * tpw, tpw)], rows_v)
        copies = [
            pltpu.async_copy(rows_v, out_hbm.at[idx_v.at[k]], sem)
            for k in range(NPK)
        ]
        for cp in copies:
            cp.wait()

    return scatter_k(table, idx3)


def _mlp_kernel(be_ref, nb_ref, x_ref, wfc_ref, bfc_ref, wproj_ref, bproj_ref,
                y_ref):
    b = pl.program_id(0)

    @pl.when(b < nb_ref[0])
    def _():
        xb = x_ref[...].astype(jnp.bfloat16)
        h = jax.lax.dot_general(xb, wfc_ref[0].astype(jnp.bfloat16),
                                (((1,), (1,)), ((), ())),
                                preferred_element_type=jnp.float32)
        h = h + bfc_ref[0]
        h = 0.5 * h * (1.0 + jax.lax.erf(h * _INV_SQRT2))
        y = jax.lax.dot_general(h.astype(jnp.bfloat16),
                                wproj_ref[0].astype(jnp.bfloat16),
                                (((1,), (1,)), ((), ())),
                                preferred_element_type=jnp.float32)
        y_ref[...] = y + bproj_ref[0]


def _expmap_hoisted(rb, y, cv, xn2, one_m_cxn2, half_c_sf, inv_sqrt_c):
    vn2 = jnp.sum(y * y, axis=-1, keepdims=True)
    arg = jnp.sqrt(half_c_sf * vn2)
    second = (inv_sqrt_c * jnp.tanh(arg) * jax.lax.rsqrt(vn2)) * y
    ip = jnp.sum(rb * second, axis=-1, keepdims=True)
    sn2 = jnp.sum(second * second, axis=-1, keepdims=True)
    a = 1.0 + 2.0 * cv * ip
    num = (a + cv * sn2) * rb + one_m_cxn2 * second
    den = a + cv * cv * xn2 * sn2
    return num / den


def _combine_kernel(yg_ref, ref_ref, w8_ref, c_ref, out_ref):
    cv = c_ref[0, 0]
    rb = ref_ref[...]
    xn2 = jnp.sum(rb * rb, axis=-1, keepdims=True)
    sf = 2.0 / (1.0 + cv * xn2)
    one_m_cxn2 = 1.0 - cv * xn2
    half_c_sf = 0.5 * cv * sf
    inv_sqrt_c = jax.lax.rsqrt(cv)
    acc = jnp.zeros((BTC, N_EMBD), dtype=jnp.float32)
    for k in range(NPK):
        z = _expmap_hoisted(rb, yg_ref[k], cv, xn2, one_m_cxn2, half_c_sf,
                            inv_sqrt_c)
        acc = acc + w8_ref[:, k:k + 1] * z
    out_ref[...] = acc


def kernel(x, reference_point, c, Wg, bg, Wfc, bfc, Wproj, bproj):
    b, t, ch = x.shape
    x_flat = x.reshape(b * t, ch)
    ref_flat = reference_point.reshape(b * t, ch)

    wgt = jnp.zeros((N_EMBD, N_EXPERTS), dtype=jnp.float32).at[:, : N_EXPERTS - 1].set(Wg.T)
    bgp = jnp.full((1, N_EXPERTS), -1e30, dtype=jnp.float32).at[0, : N_EXPERTS - 1].set(bg)

    rw, idx8, w8 = pl.pallas_call(
        _gate_kernel,
        grid=(T // BTG,),
        in_specs=[
            pl.BlockSpec((BTG, N_EMBD), lambda i: (i, 0)),
            pl.BlockSpec((N_EMBD, N_EXPERTS), lambda i: (0, 0)),
            pl.BlockSpec((1, N_EXPERTS), lambda i: (0, 0)),
        ],
        out_specs=[
            pl.BlockSpec((BTG, N_EXPERTS), lambda i: (i, 0)),
            pl.BlockSpec((BTG, NPK), lambda i: (i, 0)),
            pl.BlockSpec((BTG, NPK), lambda i: (i, 0)),
        ],
        out_shape=[
            jax.ShapeDtypeStruct((T, N_EXPERTS), jnp.float32),
            jax.ShapeDtypeStruct((T, NPK), jnp.int32),
            jax.ShapeDtypeStruct((T, NPK), jnp.float32),
        ],
    )(x_flat, wgt, bgp)

    cnt = jax.nn.one_hot(idx8, N_EXPERTS, dtype=jnp.int32).sum(axis=1)
    ranks = jnp.cumsum(cnt, axis=0) - cnt
    counts = cnt.sum(axis=0)
    nb = (counts + BT - 1) // BT
    csum_nb = jnp.cumsum(nb)
    po = (csum_nb - nb) * BT
    nblocks = csum_nb[-1].reshape(1)
    pos_full = po[None, :] + ranks
    pos8 = jnp.take_along_axis(pos_full, idx8, axis=1)
    block_expert = jnp.minimum(
        jnp.sum(jnp.arange(NBLK, dtype=jnp.int32)[:, None] >= csum_nb[None, :],
                axis=1),
        N_EXPERTS - 1).astype(jnp.int32)

    idx_s = pos8.reshape(_NW, T // _NW, NPK).transpose(0, 2, 1)
    x_sorted = _sc_scatter_rows(x_flat, idx_s, PADTOT)

    y_sorted = pl.pallas_call(
        _mlp_kernel,
        grid_spec=pltpu.PrefetchScalarGridSpec(
            num_scalar_prefetch=2,
            grid=(NBLK,),
            in_specs=[
                pl.BlockSpec((BT, N_EMBD), lambda bi, be, nbl: (bi, 0)),
                pl.BlockSpec((1, HIDDEN, N_EMBD),
                             lambda bi, be, nbl: (be[bi], 0, 0)),
                pl.BlockSpec((1, 1, HIDDEN), lambda bi, be, nbl: (be[bi], 0, 0)),
                pl.BlockSpec((1, N_EMBD, HIDDEN),
                             lambda bi, be, nbl: (be[bi], 0, 0)),
                pl.BlockSpec((1, 1, N_EMBD), lambda bi, be, nbl: (be[bi], 0, 0)),
            ],
            out_specs=pl.BlockSpec((BT, N_EMBD), lambda bi, be, nbl: (bi, 0)),
        ),
        out_shape=jax.ShapeDtypeStruct((PADTOT, N_EMBD), jnp.float32),
    )(block_expert, nblocks, x_sorted, Wfc,
      bfc.reshape(N_EXPERTS, 1, HIDDEN), Wproj,
      bproj.reshape(N_EXPERTS, 1, N_EMBD))

    idx_z = pos8.T.reshape(-1)
    zg_chunk, zg_nch = 128, (NPK * T) // (_NW * 128)
    yg = _sc_gather(y_sorted, idx_z.reshape(_NW, zg_nch, zg_chunk),
                    zg_chunk, zg_nch, NPK * T)

    out = pl.pallas_call(
        _combine_kernel,
        grid=(T // BTC,),
        in_specs=[
            pl.BlockSpec((NPK, BTC, N_EMBD), lambda i: (0, i, 0)),
            pl.BlockSpec((BTC, N_EMBD), lambda i: (i, 0)),
            pl.BlockSpec((BTC, NPK), lambda i: (i, 0)),
            pl.BlockSpec((1, 1), lambda i: (0, 0)),
        ],
        out_specs=pl.BlockSpec((BTC, N_EMBD), lambda i: (i, 0)),
        out_shape=jax.ShapeDtypeStruct((T, N_EMBD), jnp.float32),
    )(yg.reshape(NPK, T, N_EMBD), ref_flat, w8, c.reshape(1, 1))

    return (out.reshape(b, t, ch), rw)
